# baseline (speedup 1.0000x reference)
import functools

import jax
import jax.numpy as jnp
from jax import lax
from jax.experimental import pallas as pl
from jax.experimental.pallas import tpu as pltpu
from jax.experimental.pallas import tpu_sc as plsc

U = 25000
I = 25000
N = U + I
D = 64
B = 1024
EG = 800000
EH = 400000
E_TOT = EG + 2 * EH

NC = 2
NS = 16
NW = NC * NS
L = 16

CH = 2000
NCHUNKS = E_TOT // CH
KPW = NCHUNKS // NW
NV = CH // L
COMP_UNROLL = 25
G = 128
SG = 32
NSG = G // SG
CPAD = 2048
TRASH = 2 * B
ACC_ROWS = 2 * B + 1

_mesh = plsc.VectorSubcoreMesh(
    core_axis_name="c", subcore_axis_name="s", num_cores=NC, num_subcores=NS)


def _accum_body(users, items, cu_e, ci_e, erow, ecol, eval_, emb,
                out_part,
                inv, ubuf, ibuf, cubuf, cibuf,
                rowA, colA, valA, rowB, colB, valB,
                ccol0, cval0, cslot0, ccol1, cval1, cslot1,
                growA, growB, sidx0, sidx1, acc,
                semLA, semLB, semGA, semGB):
  c = lax.axis_index("c")
  s = lax.axis_index("s")
  wid = s * NC + c
  lane = lax.broadcasted_iota(jnp.int32, (L,), 0)
  trash_v = jnp.full((L,), TRASH, jnp.int32)

  pltpu.sync_copy(users, ubuf)
  pltpu.sync_copy(items, ibuf)
  pltpu.sync_copy(cu_e, cubuf)
  pltpu.sync_copy(ci_e, cibuf)

  def init_body(i, carry):
    inv[pl.ds(i * L, L)] = jnp.full((L,), -1, jnp.int32)
    return carry
  lax.fori_loop(0, N // L, init_body, 0)

  def scat_body(j, carry):
    sl = pl.ds(j * L, L)
    plsc.store_scatter(inv, [ubuf[sl]], cubuf[sl])
    plsc.store_scatter(inv, [ibuf[sl] + U], cibuf[sl])
    return carry
  lax.fori_loop(0, B // L, scat_body, 0)

  def zg_body(i, carry):
    for cc in range(D // L):
      growA[i, pl.ds(cc * L, L)] = jnp.zeros((L,), jnp.float32)
    return carry
  lax.fori_loop(0, G, zg_body, 0)
  pltpu.sync_copy(growA, acc.at[pl.ds(pl.multiple_of(s * G, G), G)])
  plsc.subcore_barrier()

  def ccol_init(i, carry):
    ccol0[pl.ds(i * L, L)] = jnp.zeros((L,), jnp.int32)
    ccol1[pl.ds(i * L, L)] = jnp.zeros((L,), jnp.int32)
    return carry
  lax.fori_loop(0, CPAD // L, ccol_init, 0)

  def chunk_base(k):
    return pl.multiple_of((wid + NW * k) * CH, 16)

  def fire_loads(base, rbuf, cbuf, vbuf, sem):
    pltpu.async_copy(erow.at[pl.ds(base, CH)], rbuf, sem)
    pltpu.async_copy(ecol.at[pl.ds(base, CH)], cbuf, sem)
    pltpu.async_copy(eval_.at[pl.ds(base, CH)], vbuf, sem)

  def wait_loads(base, rbuf, cbuf, vbuf, sem):
    pltpu.make_async_copy(erow.at[pl.ds(base, CH)], rbuf, sem).wait()
    pltpu.make_async_copy(ecol.at[pl.ds(base, CH)], cbuf, sem).wait()
    pltpu.make_async_copy(eval_.at[pl.ds(base, CH)], vbuf, sem).wait()

  def compact(rbuf, cbuf, vbuf, ccolX, cvalX, cslotX):
    ones_i = jnp.ones((L,), jnp.int32)

    def comp_body(v5, cntv):
      for u in range(COMP_UNROLL):
        sl = pl.ds((v5 * COMP_UNROLL + u) * L, L)
        s16 = plsc.load_gather(inv, [rbuf[sl]])
        m = s16 >= 0
        pos = cntv + plsc.cumsum(ones_i, mask=m) - 1
        plsc.store_scatter(ccolX, [pos], cbuf[sl], mask=m)
        plsc.store_scatter(cvalX, [pos], vbuf[sl], mask=m)
        plsc.store_scatter(cslotX, [pos], s16, mask=m)
        cntv = cntv + plsc.all_reduce_population_count(m)
      return cntv
    cntv = lax.fori_loop(0, NV // COMP_UNROLL, comp_body,
                         jnp.zeros((L,), jnp.int32))
    return jnp.max(cntv)

  def fire_group(ccolX, growX, gb, kc, semX):
    for i in range(NSG):
      @pl.when(gb + i * SG < kc)
      def _():
        pltpu.async_copy(emb.at[ccolX.at[pl.ds(gb + i * SG, SG)]],
                         growX.at[pl.ds(i * SG, SG)], semX)

  def drain_group(ccolX, growX, gb, kc, semX):
    for i in range(NSG):
      @pl.when(gb + i * SG < kc)
      def _():
        pltpu.make_async_copy(emb.at[ccolX.at[pl.ds(gb + i * SG, SG)]],
                              growX.at[pl.ds(i * SG, SG)], semX).wait()

  def stage_sidx(cslotX, sidxX, gb, kc):
    for i in range(G // L):
      sv = cslotX[pl.ds(gb + i * L, L)]
      here = gb + i * L + lane
      sidxX[pl.ds(i * L, L)] = jnp.where(here < kc, sv, trash_v)

  def weight_scatter(cvalX, sidxX, growX, gb, kc):
    nw_ = jnp.minimum(G, kc - gb)

    def w_body(r, carry3):
      vb = plsc.load_gather(cvalX, [jnp.full((L,), gb + r, jnp.int32)])
      for cc in range(D // L):
        sl = pl.ds(cc * L, L)
        growX[r, sl] = growX[r, sl] * vb
      return carry3
    lax.fori_loop(0, nw_, w_body, 0)
    pltpu.sync_copy(growX, acc.at[sidxX], add=True)

  def extra_groups(ccolX, cvalX, cslotX, sidxX, growX, kc, semX):
    n_g = (kc + (G - 1)) // G

    def eb(g, carry2):
      gb = pl.multiple_of(g * G, G)
      fire_group(ccolX, growX, gb, kc, semX)
      stage_sidx(cslotX, sidxX, gb, kc)
      drain_group(ccolX, growX, gb, kc, semX)
      weight_scatter(cvalX, sidxX, growX, gb, kc)
      return carry2
    lax.fori_loop(1, n_g, eb, 0)

  b0 = chunk_base(0)
  fire_loads(b0, rowA, colA, valA, semLA)
  wait_loads(b0, rowA, colA, valA, semLA)
  fire_loads(chunk_base(1), rowB, colB, valB, semLB)
  kc0 = compact(rowA, colA, valA, ccol0, cval0, cslot0)
  stage_sidx(cslot0, sidx0, 0, kc0)
  fire_group(ccol0, growA, 0, kc0, semGA)

  def pair_body(j, kc_ev):
    k1 = 2 * j + 1
    wait_loads(chunk_base(k1), rowB, colB, valB, semLB)
    fire_loads(chunk_base(k1 + 1), rowA, colA, valA, semLA)
    kc1 = compact(rowB, colB, valB, ccol1, cval1, cslot1)
    stage_sidx(cslot1, sidx1, 0, kc1)
    fire_group(ccol1, growB, 0, kc1, semGB)
    drain_group(ccol0, growA, 0, kc_ev, semGA)
    weight_scatter(cval0, sidx0, growA, 0, kc_ev)
    extra_groups(ccol0, cval0, cslot0, sidx0, growA, kc_ev, semGA)

    k2 = 2 * j + 2
    wait_loads(chunk_base(k2), rowA, colA, valA, semLA)

    @pl.when(k2 + 1 < KPW)
    def _():
      fire_loads(chunk_base(k2 + 1), rowB, colB, valB, semLB)
    kc2 = compact(rowA, colA, valA, ccol0, cval0, cslot0)
    stage_sidx(cslot0, sidx0, 0, kc2)
    fire_group(ccol0, growA, 0, kc2, semGA)
    drain_group(ccol1, growB, 0, kc1, semGB)
    weight_scatter(cval1, sidx1, growB, 0, kc1)
    extra_groups(ccol1, cval1, cslot1, sidx1, growB, kc1, semGB)
    return kc2

  kc_last = lax.fori_loop(0, (KPW - 1) // 2, pair_body, kc0)

  drain_group(ccol0, growA, 0, kc_last, semGA)
  weight_scatter(cval0, sidx0, growA, 0, kc_last)
  extra_groups(ccol0, cval0, cslot0, sidx0, growA, kc_last, semGA)

  plsc.subcore_barrier()
  row0 = pl.multiple_of(s * G, G)
  pltpu.sync_copy(acc.at[pl.ds(row0, G)], out_part.at[c].at[pl.ds(row0, G)])


_accum = functools.partial(
    pl.kernel,
    out_type=jax.ShapeDtypeStruct((NC, 2 * B, D), jnp.float32),
    mesh=_mesh,
    compiler_params=pltpu.CompilerParams(
        needs_layout_passes=False, use_tc_tiling_on_sc=False),
    scratch_types=[
        pltpu.VMEM((N,), jnp.int32),
        pltpu.VMEM((B,), jnp.int32),
        pltpu.VMEM((B,), jnp.int32),
        pltpu.VMEM((B,), jnp.int32),
        pltpu.VMEM((B,), jnp.int32),
        pltpu.VMEM((CH,), jnp.int32),
        pltpu.VMEM((CH,), jnp.int32),
        pltpu.VMEM((CH,), jnp.float32),
        pltpu.VMEM((CH,), jnp.int32),
        pltpu.VMEM((CH,), jnp.int32),
        pltpu.VMEM((CH,), jnp.float32),
        pltpu.VMEM((CPAD,), jnp.int32),
        pltpu.VMEM((CPAD,), jnp.float32),
        pltpu.VMEM((CPAD,), jnp.int32),
        pltpu.VMEM((CPAD,), jnp.int32),
        pltpu.VMEM((CPAD,), jnp.float32),
        pltpu.VMEM((CPAD,), jnp.int32),
        pltpu.VMEM((G, D), jnp.float32),
        pltpu.VMEM((G, D), jnp.float32),
        pltpu.VMEM((G,), jnp.int32),
        pltpu.VMEM((G,), jnp.int32),
        pltpu.VMEM_SHARED((ACC_ROWS, D), jnp.float32),
        pltpu.SemaphoreType.DMA,
        pltpu.SemaphoreType.DMA,
        pltpu.SemaphoreType.DMA,
        pltpu.SemaphoreType.DMA,
    ],
)(_accum_body)


BPW = B // NW


def _dot_body(users, items, utab, itab, p0, p1, cu_e, ci_e,
              gamma,
              u32, i32, cu32, ci32, ub, ib, p0u, p1u, p0i, p1i, gbuf, sem):
  c = lax.axis_index("c")
  s = lax.axis_index("s")
  wid = s * NC + c
  wb = pl.multiple_of(wid * BPW, BPW)

  pltpu.sync_copy(users.at[pl.ds(wb, BPW)], u32)
  pltpu.sync_copy(items.at[pl.ds(wb, BPW)], i32)
  pltpu.sync_copy(cu_e.at[pl.ds(wb, BPW)], cu32)
  pltpu.sync_copy(ci_e.at[pl.ds(wb, BPW)], ci32)

  pltpu.async_copy(utab.at[u32], ub, sem)
  pltpu.async_copy(itab.at[i32], ib, sem)
  pltpu.async_copy(p0.at[cu32], p0u, sem)
  pltpu.async_copy(p1.at[cu32], p1u, sem)
  pltpu.async_copy(p0.at[ci32], p0i, sem)
  pltpu.async_copy(p1.at[ci32], p1i, sem)
  pltpu.make_async_copy(utab.at[u32], ub, sem).wait()
  pltpu.make_async_copy(itab.at[i32], ib, sem).wait()
  pltpu.make_async_copy(p0.at[cu32], p0u, sem).wait()
  pltpu.make_async_copy(p1.at[cu32], p1u, sem).wait()
  pltpu.make_async_copy(p0.at[ci32], p0i, sem).wait()
  pltpu.make_async_copy(p1.at[ci32], p1i, sem).wait()

  lane = lax.broadcasted_iota(jnp.int32, (L,), 0)

  def half_body(j, carry):
    def b_body(b2, resv):
      b = j * L + b2
      accv = jnp.zeros((L,), jnp.float32)
      for cc in range(D // L):
        sl = pl.ds(cc * L, L)
        uv = ub[b, sl] + p0u[b, sl] + p1u[b, sl]
        iv = ib[b, sl] + p0i[b, sl] + p1i[b, sl]
        accv = accv + uv * iv
      dsum = jnp.sum(accv) * jnp.float32(1.0 / 9.0)
      return jnp.where(lane == b2, jnp.full((L,), dsum), resv)
    resv = lax.fori_loop(0, L, b_body, jnp.zeros((L,), jnp.float32))
    gbuf[pl.ds(j * L, L)] = resv
    return carry
  lax.fori_loop(0, BPW // L, half_body, 0)

  pltpu.sync_copy(gbuf, gamma.at[pl.ds(wb, BPW)])


_dot = functools.partial(
    pl.kernel,
    out_type=jax.ShapeDtypeStruct((B,), jnp.float32),
    mesh=_mesh,
    compiler_params=pltpu.CompilerParams(
        needs_layout_passes=False, use_tc_tiling_on_sc=False),
    scratch_types=[
        pltpu.VMEM((BPW,), jnp.int32),
        pltpu.VMEM((BPW,), jnp.int32),
        pltpu.VMEM((BPW,), jnp.int32),
        pltpu.VMEM((BPW,), jnp.int32),
        pltpu.VMEM((BPW, D), jnp.float32),
        pltpu.VMEM((BPW, D), jnp.float32),
        pltpu.VMEM((BPW, D), jnp.float32),
        pltpu.VMEM((BPW, D), jnp.float32),
        pltpu.VMEM((BPW, D), jnp.float32),
        pltpu.VMEM((BPW, D), jnp.float32),
        pltpu.VMEM((BPW,), jnp.float32),
        pltpu.SemaphoreType.DMA,
    ],
)(_dot_body)


def kernel(users, items, g_row, g_col, g_val, m1_row, m1_col, m1_val,
           m2_row, m2_col, m2_val, user_table, item_table):
  users = users.astype(jnp.int32)
  items = items.astype(jnp.int32)

  all_emb = jnp.concatenate([user_table, item_table], axis=0)
  erow = jnp.concatenate([g_row.astype(jnp.int32), m1_row.astype(jnp.int32),
                          m2_row.astype(jnp.int32) + U])
  ecol = jnp.concatenate([g_col.astype(jnp.int32), m1_col.astype(jnp.int32),
                          m2_col.astype(jnp.int32) + U])
  eval_ = jnp.concatenate([g_val, m1_val, m2_val])

  ar = jnp.arange(B, dtype=jnp.int32)
  cu_e = jnp.full((U,), B, jnp.int32).at[users].min(ar)[users]
  ci_e = jnp.full((I,), B, jnp.int32).at[items].min(ar)[items] + B

  partials = _accum(users, items, cu_e, ci_e, erow, ecol, eval_, all_emb)
  gamma = _dot(users, items, user_table, item_table,
               partials[0], partials[1], cu_e, ci_e)
  return gamma


# speedup vs baseline: 1.1819x; 1.1819x over previous
import functools

import jax
import jax.numpy as jnp
from jax import lax
from jax.experimental import pallas as pl
from jax.experimental.pallas import tpu as pltpu
from jax.experimental.pallas import tpu_sc as plsc

U = 25000
I = 25000
N = U + I
D = 64
B = 1024
EG = 800000
EH = 400000
E_TOT = EG + 2 * EH

NC = 2
NS = 16
NW = NC * NS
L = 16

CH = 2000
NCHUNKS = E_TOT // CH
KPW = NCHUNKS // NW
NV = CH // L
COMP_UNROLL = 25
G = 128
SG = 32
NSG = G // SG
CPAD = 2048
TRASH = 2 * B
ACC_ROWS = 2 * B + 1

_mesh = plsc.VectorSubcoreMesh(
    core_axis_name="c", subcore_axis_name="s", num_cores=NC, num_subcores=NS)


def _accum_body(users, items, erow, ecol, eval_, emb,
                out_part,
                inv, ubuf, ibuf,
                rowA, colA, valA, rowB, colB, valB,
                ccol0, cval0, cslot0, ccol1, cval1, cslot1,
                growA, growB, sidx0, sidx1, acc,
                semLA, semLB, semGA, semGB):
  c = lax.axis_index("c")
  s = lax.axis_index("s")
  wid = s * NC + c
  lane = lax.broadcasted_iota(jnp.int32, (L,), 0)
  trash_v = jnp.full((L,), TRASH, jnp.int32)

  pltpu.sync_copy(users, ubuf)
  pltpu.sync_copy(items, ibuf)

  def init_body(i, carry):
    inv[pl.ds(i * L, L)] = jnp.full((L,), -1, jnp.int32)
    return carry
  lax.fori_loop(0, N // L, init_body, 0)

  def scat_body(j, carry):
    sl = pl.ds(j * L, L)
    slot16 = j * L + lane
    plsc.store_scatter(inv, [ubuf[sl]], slot16)
    plsc.store_scatter(inv, [ibuf[sl] + U], slot16 + B)
    return carry
  lax.fori_loop(0, B // L, scat_body, 0)

  def zg_body(i, carry):
    for cc in range(D // L):
      growA[i, pl.ds(cc * L, L)] = jnp.zeros((L,), jnp.float32)
    return carry
  lax.fori_loop(0, G, zg_body, 0)
  pltpu.sync_copy(growA, acc.at[pl.ds(pl.multiple_of(s * G, G), G)])
  plsc.subcore_barrier()

  def ccol_init(i, carry):
    ccol0[pl.ds(i * L, L)] = jnp.zeros((L,), jnp.int32)
    ccol1[pl.ds(i * L, L)] = jnp.zeros((L,), jnp.int32)
    return carry
  lax.fori_loop(0, CPAD // L, ccol_init, 0)

  def chunk_base(k):
    return pl.multiple_of((wid + NW * k) * CH, 16)

  def fire_loads(base, rbuf, cbuf, vbuf, sem):
    pltpu.async_copy(erow.at[pl.ds(base, CH)], rbuf, sem)
    pltpu.async_copy(ecol.at[pl.ds(base, CH)], cbuf, sem)
    pltpu.async_copy(eval_.at[pl.ds(base, CH)], vbuf, sem)

  def wait_loads(base, rbuf, cbuf, vbuf, sem):
    pltpu.make_async_copy(erow.at[pl.ds(base, CH)], rbuf, sem).wait()
    pltpu.make_async_copy(ecol.at[pl.ds(base, CH)], cbuf, sem).wait()
    pltpu.make_async_copy(eval_.at[pl.ds(base, CH)], vbuf, sem).wait()

  def compact(rbuf, cbuf, vbuf, ccolX, cvalX, cslotX):
    ones_i = jnp.ones((L,), jnp.int32)

    def comp_body(v5, cntv):
      for u in range(COMP_UNROLL):
        sl = pl.ds((v5 * COMP_UNROLL + u) * L, L)
        s16 = plsc.load_gather(inv, [rbuf[sl]])
        m = s16 >= 0
        pos = cntv + plsc.cumsum(ones_i, mask=m) - 1
        plsc.store_scatter(ccolX, [pos], cbuf[sl], mask=m)
        plsc.store_scatter(cvalX, [pos], vbuf[sl], mask=m)
        plsc.store_scatter(cslotX, [pos], s16, mask=m)
        cntv = cntv + plsc.all_reduce_population_count(m)
      return cntv
    cntv = lax.fori_loop(0, NV // COMP_UNROLL, comp_body,
                         jnp.zeros((L,), jnp.int32))
    return jnp.max(cntv)

  def fire_group(ccolX, growX, gb, kc, semX):
    for i in range(NSG):
      @pl.when(gb + i * SG < kc)
      def _():
        pltpu.async_copy(emb.at[ccolX.at[pl.ds(gb + i * SG, SG)]],
                         growX.at[pl.ds(i * SG, SG)], semX)

  def drain_group(ccolX, growX, gb, kc, semX):
    for i in range(NSG):
      @pl.when(gb + i * SG < kc)
      def _():
        pltpu.make_async_copy(emb.at[ccolX.at[pl.ds(gb + i * SG, SG)]],
                              growX.at[pl.ds(i * SG, SG)], semX).wait()

  def stage_sidx(cslotX, sidxX, gb, kc):
    for i in range(G // L):
      sv = cslotX[pl.ds(gb + i * L, L)]
      here = gb + i * L + lane
      sidxX[pl.ds(i * L, L)] = jnp.where(here < kc, sv, trash_v)

  def weight_scatter(cvalX, sidxX, growX, gb, kc):
    nw_ = jnp.minimum(G, kc - gb)

    def w_body(r, carry3):
      vb = plsc.load_gather(cvalX, [jnp.full((L,), gb + r, jnp.int32)])
      for cc in range(D // L):
        sl = pl.ds(cc * L, L)
        growX[r, sl] = growX[r, sl] * vb
      return carry3
    lax.fori_loop(0, nw_, w_body, 0)
    pltpu.sync_copy(growX, acc.at[sidxX], add=True)

  def extra_groups(ccolX, cvalX, cslotX, sidxX, growX, kc, semX):
    n_g = (kc + (G - 1)) // G

    def eb(g, carry2):
      gb = pl.multiple_of(g * G, G)
      fire_group(ccolX, growX, gb, kc, semX)
      stage_sidx(cslotX, sidxX, gb, kc)
      drain_group(ccolX, growX, gb, kc, semX)
      weight_scatter(cvalX, sidxX, growX, gb, kc)
      return carry2
    lax.fori_loop(1, n_g, eb, 0)

  b0 = chunk_base(0)
  fire_loads(b0, rowA, colA, valA, semLA)
  wait_loads(b0, rowA, colA, valA, semLA)
  fire_loads(chunk_base(1), rowB, colB, valB, semLB)
  kc0 = compact(rowA, colA, valA, ccol0, cval0, cslot0)
  stage_sidx(cslot0, sidx0, 0, kc0)
  fire_group(ccol0, growA, 0, kc0, semGA)

  def pair_body(j, kc_ev):
    k1 = 2 * j + 1
    wait_loads(chunk_base(k1), rowB, colB, valB, semLB)
    fire_loads(chunk_base(k1 + 1), rowA, colA, valA, semLA)
    kc1 = compact(rowB, colB, valB, ccol1, cval1, cslot1)
    stage_sidx(cslot1, sidx1, 0, kc1)
    fire_group(ccol1, growB, 0, kc1, semGB)
    drain_group(ccol0, growA, 0, kc_ev, semGA)
    weight_scatter(cval0, sidx0, growA, 0, kc_ev)
    extra_groups(ccol0, cval0, cslot0, sidx0, growA, kc_ev, semGA)

    k2 = 2 * j + 2
    wait_loads(chunk_base(k2), rowA, colA, valA, semLA)

    @pl.when(k2 + 1 < KPW)
    def _():
      fire_loads(chunk_base(k2 + 1), rowB, colB, valB, semLB)
    kc2 = compact(rowA, colA, valA, ccol0, cval0, cslot0)
    stage_sidx(cslot0, sidx0, 0, kc2)
    fire_group(ccol0, growA, 0, kc2, semGA)
    drain_group(ccol1, growB, 0, kc1, semGB)
    weight_scatter(cval1, sidx1, growB, 0, kc1)
    extra_groups(ccol1, cval1, cslot1, sidx1, growB, kc1, semGB)
    return kc2

  kc_last = lax.fori_loop(0, (KPW - 1) // 2, pair_body, kc0)

  drain_group(ccol0, growA, 0, kc_last, semGA)
  weight_scatter(cval0, sidx0, growA, 0, kc_last)
  extra_groups(ccol0, cval0, cslot0, sidx0, growA, kc_last, semGA)

  plsc.subcore_barrier()
  row0 = pl.multiple_of(s * G, G)
  pltpu.sync_copy(acc.at[pl.ds(row0, G)], out_part.at[c].at[pl.ds(row0, G)])


_accum = functools.partial(
    pl.kernel,
    out_type=jax.ShapeDtypeStruct((NC, 2 * B, D), jnp.float32),
    mesh=_mesh,
    compiler_params=pltpu.CompilerParams(
        needs_layout_passes=False, use_tc_tiling_on_sc=False),
    scratch_types=[
        pltpu.VMEM((N,), jnp.int32),
        pltpu.VMEM((B,), jnp.int32),
        pltpu.VMEM((B,), jnp.int32),
        pltpu.VMEM((CH,), jnp.int32),
        pltpu.VMEM((CH,), jnp.int32),
        pltpu.VMEM((CH,), jnp.float32),
        pltpu.VMEM((CH,), jnp.int32),
        pltpu.VMEM((CH,), jnp.int32),
        pltpu.VMEM((CH,), jnp.float32),
        pltpu.VMEM((CPAD,), jnp.int32),
        pltpu.VMEM((CPAD,), jnp.float32),
        pltpu.VMEM((CPAD,), jnp.int32),
        pltpu.VMEM((CPAD,), jnp.int32),
        pltpu.VMEM((CPAD,), jnp.float32),
        pltpu.VMEM((CPAD,), jnp.int32),
        pltpu.VMEM((G, D), jnp.float32),
        pltpu.VMEM((G, D), jnp.float32),
        pltpu.VMEM((G,), jnp.int32),
        pltpu.VMEM((G,), jnp.int32),
        pltpu.VMEM_SHARED((ACC_ROWS, D), jnp.float32),
        pltpu.SemaphoreType.DMA,
        pltpu.SemaphoreType.DMA,
        pltpu.SemaphoreType.DMA,
        pltpu.SemaphoreType.DMA,
    ],
)(_accum_body)


BPW = B // NW


def _dot_body(users, items, utab, itab, p0, p1,
              gamma,
              inv, ubuf, ibuf, u32, i32, cu32, ci32,
              ub, ib, p0u, p1u, p0i, p1i, gbuf, sem):
  c = lax.axis_index("c")
  s = lax.axis_index("s")
  wid = s * NC + c
  wb = pl.multiple_of(wid * BPW, BPW)
  lane0 = lax.broadcasted_iota(jnp.int32, (L,), 0)

  pltpu.sync_copy(users, ubuf)
  pltpu.sync_copy(items, ibuf)

  def init_body(i, carry):
    inv[pl.ds(i * L, L)] = jnp.full((L,), -1, jnp.int32)
    return carry
  lax.fori_loop(0, N // L, init_body, 0)

  def scat_body(j, carry):
    sl = pl.ds(j * L, L)
    slot16 = j * L + lane0
    plsc.store_scatter(inv, [ubuf[sl]], slot16)
    plsc.store_scatter(inv, [ibuf[sl] + U], slot16 + B)
    return carry
  lax.fori_loop(0, B // L, scat_body, 0)

  pltpu.sync_copy(users.at[pl.ds(wb, BPW)], u32)
  pltpu.sync_copy(items.at[pl.ds(wb, BPW)], i32)
  for i in range(BPW // L):
    sl = pl.ds(i * L, L)
    cu32[sl] = plsc.load_gather(inv, [u32[sl]])
    ci32[sl] = plsc.load_gather(inv, [i32[sl] + U])

  pltpu.async_copy(utab.at[u32], ub, sem)
  pltpu.async_copy(itab.at[i32], ib, sem)
  pltpu.async_copy(p0.at[cu32], p0u, sem)
  pltpu.async_copy(p1.at[cu32], p1u, sem)
  pltpu.async_copy(p0.at[ci32], p0i, sem)
  pltpu.async_copy(p1.at[ci32], p1i, sem)
  pltpu.make_async_copy(utab.at[u32], ub, sem).wait()
  pltpu.make_async_copy(itab.at[i32], ib, sem).wait()
  pltpu.make_async_copy(p0.at[cu32], p0u, sem).wait()
  pltpu.make_async_copy(p1.at[cu32], p1u, sem).wait()
  pltpu.make_async_copy(p0.at[ci32], p0i, sem).wait()
  pltpu.make_async_copy(p1.at[ci32], p1i, sem).wait()

  lane = lax.broadcasted_iota(jnp.int32, (L,), 0)

  def half_body(j, carry):
    def b_body(b2, resv):
      b = j * L + b2
      accv = jnp.zeros((L,), jnp.float32)
      for cc in range(D // L):
        sl = pl.ds(cc * L, L)
        uv = ub[b, sl] + p0u[b, sl] + p1u[b, sl]
        iv = ib[b, sl] + p0i[b, sl] + p1i[b, sl]
        accv = accv + uv * iv
      dsum = jnp.sum(accv) * jnp.float32(1.0 / 9.0)
      return jnp.where(lane == b2, jnp.full((L,), dsum), resv)
    resv = lax.fori_loop(0, L, b_body, jnp.zeros((L,), jnp.float32))
    gbuf[pl.ds(j * L, L)] = resv
    return carry
  lax.fori_loop(0, BPW // L, half_body, 0)

  pltpu.sync_copy(gbuf, gamma.at[pl.ds(wb, BPW)])


_dot = functools.partial(
    pl.kernel,
    out_type=jax.ShapeDtypeStruct((B,), jnp.float32),
    mesh=_mesh,
    compiler_params=pltpu.CompilerParams(
        needs_layout_passes=False, use_tc_tiling_on_sc=False),
    scratch_types=[
        pltpu.VMEM((N,), jnp.int32),
        pltpu.VMEM((B,), jnp.int32),
        pltpu.VMEM((B,), jnp.int32),
        pltpu.VMEM((BPW,), jnp.int32),
        pltpu.VMEM((BPW,), jnp.int32),
        pltpu.VMEM((BPW,), jnp.int32),
        pltpu.VMEM((BPW,), jnp.int32),
        pltpu.VMEM((BPW, D), jnp.float32),
        pltpu.VMEM((BPW, D), jnp.float32),
        pltpu.VMEM((BPW, D), jnp.float32),
        pltpu.VMEM((BPW, D), jnp.float32),
        pltpu.VMEM((BPW, D), jnp.float32),
        pltpu.VMEM((BPW, D), jnp.float32),
        pltpu.VMEM((BPW,), jnp.float32),
        pltpu.SemaphoreType.DMA,
    ],
)(_dot_body)


def kernel(users, items, g_row, g_col, g_val, m1_row, m1_col, m1_val,
           m2_row, m2_col, m2_val, user_table, item_table):
  users = users.astype(jnp.int32)
  items = items.astype(jnp.int32)

  all_emb = jnp.concatenate([user_table, item_table], axis=0)
  erow = jnp.concatenate([g_row.astype(jnp.int32), m1_row.astype(jnp.int32),
                          m2_row.astype(jnp.int32) + U])
  ecol = jnp.concatenate([g_col.astype(jnp.int32), m1_col.astype(jnp.int32),
                          m2_col.astype(jnp.int32) + U])
  eval_ = jnp.concatenate([g_val, m1_val, m2_val])

  partials = _accum(users, items, erow, ecol, eval_, all_emb)
  gamma = _dot(users, items, user_table, item_table,
               partials[0], partials[1])
  return gamma


# speedup vs baseline: 1.2445x; 1.0529x over previous
import functools

import jax
import jax.numpy as jnp
from jax import lax
from jax.experimental import pallas as pl
from jax.experimental.pallas import tpu as pltpu
from jax.experimental.pallas import tpu_sc as plsc

U = 25000
I = 25000
N = U + I
D = 64
B = 1024
EG = 800000
EH = 400000
E_TOT = EG + 2 * EH

NC = 2
NS = 16
NW = NC * NS
L = 16

CH = 2000
NCHUNKS = E_TOT // CH
KPW = NCHUNKS // NW
NV = CH // L
COMP_UNROLL = 25
G = 128
SG = 32
NSG = G // SG
CPAD = 2048
TRASH = 2 * B
ACC_ROWS = 2 * B + 1

_mesh = plsc.VectorSubcoreMesh(
    core_axis_name="c", subcore_axis_name="s", num_cores=NC, num_subcores=NS)


def _accum_body(users, items, grow_, gcol_, gval_, m1row_, m1col_, m1val_,
                m2row_, m2col_, m2val_, emb,
                out_part,
                inv, ubuf, ibuf,
                rowA, colA, valA, rowB, colB, valB,
                ccol0, cval0, cslot0, ccol1, cval1, cslot1,
                growA, growB, sidx0, sidx1, acc,
                semLA, semLB, semGA, semGB):
  c = lax.axis_index("c")
  s = lax.axis_index("s")
  wid = s * NC + c
  lane = lax.broadcasted_iota(jnp.int32, (L,), 0)
  trash_v = jnp.full((L,), TRASH, jnp.int32)

  pltpu.sync_copy(users, ubuf)
  pltpu.sync_copy(items, ibuf)

  def init_body(i, carry):
    inv[pl.ds(i * L, L)] = jnp.full((L,), -1, jnp.int32)
    return carry
  lax.fori_loop(0, N // L, init_body, 0)

  def scat_body(j, carry):
    sl = pl.ds(j * L, L)
    slot16 = j * L + lane
    plsc.store_scatter(inv, [ubuf[sl]], slot16)
    plsc.store_scatter(inv, [ibuf[sl] + U], slot16 + B)
    return carry
  lax.fori_loop(0, B // L, scat_body, 0)

  def zg_body(i, carry):
    for cc in range(D // L):
      growA[i, pl.ds(cc * L, L)] = jnp.zeros((L,), jnp.float32)
    return carry
  lax.fori_loop(0, G, zg_body, 0)
  pltpu.sync_copy(growA, acc.at[pl.ds(pl.multiple_of(s * G, G), G)])
  plsc.subcore_barrier()

  def ccol_init(i, carry):
    ccol0[pl.ds(i * L, L)] = jnp.zeros((L,), jnp.int32)
    ccol1[pl.ds(i * L, L)] = jnp.zeros((L,), jnp.int32)
    return carry
  lax.fori_loop(0, CPAD // L, ccol_init, 0)

  NG_CH = EG // CH
  NH_CH = EH // CH

  def fire_loads(k, rbuf, cbuf, vbuf, sem):
    t = wid + NW * k

    @pl.when(t < NG_CH)
    def _():
      base = pl.multiple_of(t * CH, 16)
      pltpu.async_copy(grow_.at[pl.ds(base, CH)], rbuf, sem)
      pltpu.async_copy(gcol_.at[pl.ds(base, CH)], cbuf, sem)
      pltpu.async_copy(gval_.at[pl.ds(base, CH)], vbuf, sem)

    @pl.when((t >= NG_CH) & (t < NG_CH + NH_CH))
    def _():
      base = pl.multiple_of((t - NG_CH) * CH, 16)
      pltpu.async_copy(m1row_.at[pl.ds(base, CH)], rbuf, sem)
      pltpu.async_copy(m1col_.at[pl.ds(base, CH)], cbuf, sem)
      pltpu.async_copy(m1val_.at[pl.ds(base, CH)], vbuf, sem)

    @pl.when(t >= NG_CH + NH_CH)
    def _():
      base = pl.multiple_of((t - NG_CH - NH_CH) * CH, 16)
      pltpu.async_copy(m2row_.at[pl.ds(base, CH)], rbuf, sem)
      pltpu.async_copy(m2col_.at[pl.ds(base, CH)], cbuf, sem)
      pltpu.async_copy(m2val_.at[pl.ds(base, CH)], vbuf, sem)

  def wait_loads(k, rbuf, cbuf, vbuf, sem):
    t = wid + NW * k

    @pl.when(t < NG_CH)
    def _():
      base = pl.multiple_of(t * CH, 16)
      pltpu.make_async_copy(grow_.at[pl.ds(base, CH)], rbuf, sem).wait()
      pltpu.make_async_copy(gcol_.at[pl.ds(base, CH)], cbuf, sem).wait()
      pltpu.make_async_copy(gval_.at[pl.ds(base, CH)], vbuf, sem).wait()

    @pl.when((t >= NG_CH) & (t < NG_CH + NH_CH))
    def _():
      base = pl.multiple_of((t - NG_CH) * CH, 16)
      pltpu.make_async_copy(m1row_.at[pl.ds(base, CH)], rbuf, sem).wait()
      pltpu.make_async_copy(m1col_.at[pl.ds(base, CH)], cbuf, sem).wait()
      pltpu.make_async_copy(m1val_.at[pl.ds(base, CH)], vbuf, sem).wait()

    @pl.when(t >= NG_CH + NH_CH)
    def _():
      base = pl.multiple_of((t - NG_CH - NH_CH) * CH, 16)
      pltpu.make_async_copy(m2row_.at[pl.ds(base, CH)], rbuf, sem).wait()
      pltpu.make_async_copy(m2col_.at[pl.ds(base, CH)], cbuf, sem).wait()
      pltpu.make_async_copy(m2val_.at[pl.ds(base, CH)], vbuf, sem).wait()

  def compact(rbuf, cbuf, vbuf, ccolX, cvalX, cslotX):
    ones_i = jnp.ones((L,), jnp.int32)

    def comp_body(v5, cntv):
      for u in range(COMP_UNROLL):
        sl = pl.ds((v5 * COMP_UNROLL + u) * L, L)
        s16 = plsc.load_gather(inv, [rbuf[sl]])
        m = s16 >= 0
        pos = cntv + plsc.cumsum(ones_i, mask=m) - 1
        plsc.store_scatter(ccolX, [pos], cbuf[sl], mask=m)
        plsc.store_scatter(cvalX, [pos], vbuf[sl], mask=m)
        plsc.store_scatter(cslotX, [pos], s16, mask=m)
        cntv = cntv + plsc.all_reduce_population_count(m)
      return cntv
    cntv = lax.fori_loop(0, NV // COMP_UNROLL, comp_body,
                         jnp.zeros((L,), jnp.int32))
    return jnp.max(cntv)

  def fire_group(ccolX, growX, gb, kc, semX):
    for i in range(NSG):
      @pl.when(gb + i * SG < kc)
      def _():
        pltpu.async_copy(emb.at[ccolX.at[pl.ds(gb + i * SG, SG)]],
                         growX.at[pl.ds(i * SG, SG)], semX)

  def drain_group(ccolX, growX, gb, kc, semX):
    for i in range(NSG):
      @pl.when(gb + i * SG < kc)
      def _():
        pltpu.make_async_copy(emb.at[ccolX.at[pl.ds(gb + i * SG, SG)]],
                              growX.at[pl.ds(i * SG, SG)], semX).wait()

  def stage_sidx(cslotX, sidxX, gb, kc):
    for i in range(G // L):
      sv = cslotX[pl.ds(gb + i * L, L)]
      here = gb + i * L + lane
      sidxX[pl.ds(i * L, L)] = jnp.where(here < kc, sv, trash_v)

  def weight_scatter(cvalX, sidxX, growX, gb, kc):
    nw_ = jnp.minimum(G, kc - gb)

    def w_body(r, carry3):
      vb = plsc.load_gather(cvalX, [jnp.full((L,), gb + r, jnp.int32)])
      for cc in range(D // L):
        sl = pl.ds(cc * L, L)
        growX[r, sl] = growX[r, sl] * vb
      return carry3
    lax.fori_loop(0, nw_, w_body, 0)
    pltpu.sync_copy(growX, acc.at[sidxX], add=True)

  def extra_groups(ccolX, cvalX, cslotX, sidxX, growX, kc, semX):
    n_g = (kc + (G - 1)) // G

    def eb(g, carry2):
      gb = pl.multiple_of(g * G, G)
      fire_group(ccolX, growX, gb, kc, semX)
      stage_sidx(cslotX, sidxX, gb, kc)
      drain_group(ccolX, growX, gb, kc, semX)
      weight_scatter(cvalX, sidxX, growX, gb, kc)
      return carry2
    lax.fori_loop(1, n_g, eb, 0)

  fire_loads(0, rowA, colA, valA, semLA)
  wait_loads(0, rowA, colA, valA, semLA)
  fire_loads(1, rowB, colB, valB, semLB)
  kc0 = compact(rowA, colA, valA, ccol0, cval0, cslot0)
  stage_sidx(cslot0, sidx0, 0, kc0)
  fire_group(ccol0, growA, 0, kc0, semGA)

  def pair_body(j, kc_ev):
    k1 = 2 * j + 1
    wait_loads(k1, rowB, colB, valB, semLB)
    fire_loads(k1 + 1, rowA, colA, valA, semLA)
    kc1 = compact(rowB, colB, valB, ccol1, cval1, cslot1)
    stage_sidx(cslot1, sidx1, 0, kc1)
    fire_group(ccol1, growB, 0, kc1, semGB)
    drain_group(ccol0, growA, 0, kc_ev, semGA)
    weight_scatter(cval0, sidx0, growA, 0, kc_ev)
    extra_groups(ccol0, cval0, cslot0, sidx0, growA, kc_ev, semGA)

    k2 = 2 * j + 2
    wait_loads(k2, rowA, colA, valA, semLA)

    @pl.when(k2 + 1 < KPW)
    def _():
      fire_loads(k2 + 1, rowB, colB, valB, semLB)
    kc2 = compact(rowA, colA, valA, ccol0, cval0, cslot0)
    stage_sidx(cslot0, sidx0, 0, kc2)
    fire_group(ccol0, growA, 0, kc2, semGA)
    drain_group(ccol1, growB, 0, kc1, semGB)
    weight_scatter(cval1, sidx1, growB, 0, kc1)
    extra_groups(ccol1, cval1, cslot1, sidx1, growB, kc1, semGB)
    return kc2

  kc_last = lax.fori_loop(0, (KPW - 1) // 2, pair_body, kc0)

  drain_group(ccol0, growA, 0, kc_last, semGA)
  weight_scatter(cval0, sidx0, growA, 0, kc_last)
  extra_groups(ccol0, cval0, cslot0, sidx0, growA, kc_last, semGA)

  plsc.subcore_barrier()
  row0 = pl.multiple_of(s * G, G)
  pltpu.sync_copy(acc.at[pl.ds(row0, G)], out_part.at[c].at[pl.ds(row0, G)])


_accum = functools.partial(
    pl.kernel,
    out_type=jax.ShapeDtypeStruct((NC, 2 * B, D), jnp.float32),
    mesh=_mesh,
    compiler_params=pltpu.CompilerParams(
        needs_layout_passes=False, use_tc_tiling_on_sc=False),
    scratch_types=[
        pltpu.VMEM((N,), jnp.int32),
        pltpu.VMEM((B,), jnp.int32),
        pltpu.VMEM((B,), jnp.int32),
        pltpu.VMEM((CH,), jnp.int32),
        pltpu.VMEM((CH,), jnp.int32),
        pltpu.VMEM((CH,), jnp.float32),
        pltpu.VMEM((CH,), jnp.int32),
        pltpu.VMEM((CH,), jnp.int32),
        pltpu.VMEM((CH,), jnp.float32),
        pltpu.VMEM((CPAD,), jnp.int32),
        pltpu.VMEM((CPAD,), jnp.float32),
        pltpu.VMEM((CPAD,), jnp.int32),
        pltpu.VMEM((CPAD,), jnp.int32),
        pltpu.VMEM((CPAD,), jnp.float32),
        pltpu.VMEM((CPAD,), jnp.int32),
        pltpu.VMEM((G, D), jnp.float32),
        pltpu.VMEM((G, D), jnp.float32),
        pltpu.VMEM((G,), jnp.int32),
        pltpu.VMEM((G,), jnp.int32),
        pltpu.VMEM_SHARED((ACC_ROWS, D), jnp.float32),
        pltpu.SemaphoreType.DMA,
        pltpu.SemaphoreType.DMA,
        pltpu.SemaphoreType.DMA,
        pltpu.SemaphoreType.DMA,
    ],
)(_accum_body)


BPW = B // NW


def _dot_body(users, items, utab, itab, p0, p1,
              gamma,
              inv, ubuf, ibuf, u32, i32, cu32, ci32,
              ub, ib, p0u, p1u, p0i, p1i, gbuf, sem):
  c = lax.axis_index("c")
  s = lax.axis_index("s")
  wid = s * NC + c
  wb = pl.multiple_of(wid * BPW, BPW)
  lane0 = lax.broadcasted_iota(jnp.int32, (L,), 0)

  pltpu.sync_copy(users, ubuf)
  pltpu.sync_copy(items, ibuf)

  def init_body(i, carry):
    inv[pl.ds(i * L, L)] = jnp.full((L,), -1, jnp.int32)
    return carry
  lax.fori_loop(0, N // L, init_body, 0)

  def scat_body(j, carry):
    sl = pl.ds(j * L, L)
    slot16 = j * L + lane0
    plsc.store_scatter(inv, [ubuf[sl]], slot16)
    plsc.store_scatter(inv, [ibuf[sl] + U], slot16 + B)
    return carry
  lax.fori_loop(0, B // L, scat_body, 0)

  pltpu.sync_copy(users.at[pl.ds(wb, BPW)], u32)
  pltpu.sync_copy(items.at[pl.ds(wb, BPW)], i32)
  for i in range(BPW // L):
    sl = pl.ds(i * L, L)
    cu32[sl] = plsc.load_gather(inv, [u32[sl]])
    ci32[sl] = plsc.load_gather(inv, [i32[sl] + U])

  pltpu.async_copy(utab.at[u32], ub, sem)
  pltpu.async_copy(itab.at[i32], ib, sem)
  pltpu.async_copy(p0.at[cu32], p0u, sem)
  pltpu.async_copy(p1.at[cu32], p1u, sem)
  pltpu.async_copy(p0.at[ci32], p0i, sem)
  pltpu.async_copy(p1.at[ci32], p1i, sem)
  pltpu.make_async_copy(utab.at[u32], ub, sem).wait()
  pltpu.make_async_copy(itab.at[i32], ib, sem).wait()
  pltpu.make_async_copy(p0.at[cu32], p0u, sem).wait()
  pltpu.make_async_copy(p1.at[cu32], p1u, sem).wait()
  pltpu.make_async_copy(p0.at[ci32], p0i, sem).wait()
  pltpu.make_async_copy(p1.at[ci32], p1i, sem).wait()

  lane = lax.broadcasted_iota(jnp.int32, (L,), 0)

  def half_body(j, carry):
    def b_body(b2, resv):
      b = j * L + b2
      accv = jnp.zeros((L,), jnp.float32)
      for cc in range(D // L):
        sl = pl.ds(cc * L, L)
        uv = ub[b, sl] + p0u[b, sl] + p1u[b, sl]
        iv = ib[b, sl] + p0i[b, sl] + p1i[b, sl]
        accv = accv + uv * iv
      dsum = jnp.sum(accv) * jnp.float32(1.0 / 9.0)
      return jnp.where(lane == b2, jnp.full((L,), dsum), resv)
    resv = lax.fori_loop(0, L, b_body, jnp.zeros((L,), jnp.float32))
    gbuf[pl.ds(j * L, L)] = resv
    return carry
  lax.fori_loop(0, BPW // L, half_body, 0)

  pltpu.sync_copy(gbuf, gamma.at[pl.ds(wb, BPW)])


_dot = functools.partial(
    pl.kernel,
    out_type=jax.ShapeDtypeStruct((B,), jnp.float32),
    mesh=_mesh,
    compiler_params=pltpu.CompilerParams(
        needs_layout_passes=False, use_tc_tiling_on_sc=False),
    scratch_types=[
        pltpu.VMEM((N,), jnp.int32),
        pltpu.VMEM((B,), jnp.int32),
        pltpu.VMEM((B,), jnp.int32),
        pltpu.VMEM((BPW,), jnp.int32),
        pltpu.VMEM((BPW,), jnp.int32),
        pltpu.VMEM((BPW,), jnp.int32),
        pltpu.VMEM((BPW,), jnp.int32),
        pltpu.VMEM((BPW, D), jnp.float32),
        pltpu.VMEM((BPW, D), jnp.float32),
        pltpu.VMEM((BPW, D), jnp.float32),
        pltpu.VMEM((BPW, D), jnp.float32),
        pltpu.VMEM((BPW, D), jnp.float32),
        pltpu.VMEM((BPW, D), jnp.float32),
        pltpu.VMEM((BPW,), jnp.float32),
        pltpu.SemaphoreType.DMA,
    ],
)(_dot_body)


def kernel(users, items, g_row, g_col, g_val, m1_row, m1_col, m1_val,
           m2_row, m2_col, m2_val, user_table, item_table):
  users = users.astype(jnp.int32)
  items = items.astype(jnp.int32)

  all_emb = jnp.concatenate([user_table, item_table], axis=0)
  m2r = m2_row.astype(jnp.int32) + U
  m2c = m2_col.astype(jnp.int32) + U

  partials = _accum(users, items,
                    g_row.astype(jnp.int32), g_col.astype(jnp.int32), g_val,
                    m1_row.astype(jnp.int32), m1_col.astype(jnp.int32), m1_val,
                    m2r, m2c, m2_val, all_emb)
  gamma = _dot(users, items, user_table, item_table,
               partials[0], partials[1])
  return gamma
